# Initial kernel scaffold; baseline (speedup 1.0000x reference)
import jax
import jax.numpy as jnp
from jax.experimental import pallas as pl


def kernel(xyz0, xyz1, xyz2, x0, x1, x2):
    raise NotImplementedError("write your pallas kernel here")


# speedup vs baseline: 25.5552x; 25.5552x over previous
import functools

import jax
import jax.numpy as jnp
from jax import lax
from jax.experimental import pallas as pl
from jax.experimental.pallas import tpu as pltpu

_K = 3


def _propagate_body(qT_ref, src_ref, p2_ref, out_ref, *, S):
    qf = qT_ref[0]
    sf = src_ref[0]
    qx = qf[0:1, :]
    qy = qf[1:2, :]
    qz = qf[2:3, :]
    sx = sf[:, 0:1]
    sy = sf[:, 1:2]
    sz = sf[:, 2:3]
    q2 = qx * qx + qy * qy + qz * qz
    s2 = sx * sx + sy * sy + sz * sz
    m = jnp.dot(sf, qf, preferred_element_type=jnp.float32,
                precision=lax.Precision.DEFAULT)
    d = -2.0 * m
    d = d + q2
    d = d + s2

    iota = lax.broadcasted_iota(jnp.int32, d.shape, 0)
    dd = d
    mins = []
    idxs = []
    for _ in range(_K):
        mk = jnp.min(dd, axis=0, keepdims=True)
        ik = jnp.min(jnp.where(dd <= mk, iota, S), axis=0, keepdims=True)
        dd = jnp.where(iota == ik, jnp.inf, dd)
        mins.append(mk)
        idxs.append(ik)

    recips = [1.0 / (m_ + 1e-8) for m_ in mins]
    norm = (recips[0] + recips[1]) + recips[2]
    wt = jnp.zeros(d.shape, jnp.float32)
    for r, ik in zip(recips, idxs):
        wt = jnp.where(iota == ik, r / norm, wt)

    out_ref[0] = jnp.dot(p2_ref[0], wt, preferred_element_type=jnp.float32)


def _propagate_tc(qT, src, p2T, *, nblk):
    B, _, N = qT.shape
    S = src.shape[1]
    D2 = p2T.shape[1]
    grid = (B, N // nblk)
    return pl.pallas_call(
        functools.partial(_propagate_body, S=S),
        grid=grid,
        in_specs=[
            pl.BlockSpec((1, 3, nblk), lambda b, j: (b, 0, j)),
            pl.BlockSpec((1, S, 3), lambda b, j: (b, 0, 0)),
            pl.BlockSpec((1, D2, S), lambda b, j: (b, 0, 0)),
        ],
        out_specs=pl.BlockSpec((1, D2, nblk), lambda b, j: (b, 0, j)),
        out_shape=jax.ShapeDtypeStruct((B, D2, N), jnp.float32),
        compiler_params=pltpu.CompilerParams(
            dimension_semantics=("parallel", "arbitrary"),
        ),
    )(qT, src, p2T)


def kernel(xyz0, xyz1, xyz2, x0, x1, x2):
    q1T = jnp.transpose(xyz1, (0, 2, 1))
    interp1T = _propagate_tc(q1T, xyz2, x2, nblk=256)
    f1T = jnp.concatenate([x1, interp1T], axis=1)

    q0T = jnp.transpose(xyz0, (0, 2, 1))
    interp2T = _propagate_tc(q0T, xyz1, f1T, nblk=256)
    return jnp.concatenate([x0, interp2T], axis=1)
